# baseline (speedup 1.0000x reference)
import functools

import jax
import jax.numpy as jnp
from jax import lax
from jax.experimental import pallas as pl
from jax.experimental.pallas import tpu as pltpu

EPS = 1e-5


def _conv_stats_kernel(x_ref, scale_ref, shift_ref, w_ref,
                       y_ref, s_ref, q_ref,
                       xpad, xstk, *, H, W, TILE, PADL, apply_act):
    C = x_ref.shape[1]
    HW = H * W

    x = x_ref[0]
    if apply_act:
        x = jnp.maximum(x * scale_ref[...] + shift_ref[...], 0.0)
    xb = x.astype(jnp.bfloat16)

    xpad[:, :PADL] = jnp.zeros((C, PADL), jnp.bfloat16)
    xpad[:, PADL + HW:] = jnp.zeros((C, PADL), jnp.bfloat16)
    xpad[:, PADL:PADL + HW] = xb

    SW = 2 * PADL + HW - 2 * W
    for kh in range(3):
        xstk[kh * C:(kh + 1) * C, :] = xpad[:, kh * W:kh * W + SW]

    j = lax.broadcasted_iota(jnp.int32, (1, TILE), 1) % W
    m0 = (j != 0).astype(jnp.bfloat16)
    m2 = (j != W - 1).astype(jnp.bfloat16)

    st = jnp.zeros((C, 1), jnp.float32)
    qt = jnp.zeros((C, 1), jnp.float32)
    for t in range(HW // TILE):
        base = t * TILE
        acc = None
        for kw in range(3):
            off = (PADL - W) + base + (kw - 1)
            o = xstk[:, off:off + TILE]
            if kw == 0:
                o = o * m0
            elif kw == 2:
                o = o * m2
            d = jnp.dot(w_ref[kw], o, preferred_element_type=jnp.float32)
            acc = d if acc is None else acc + d
        y_ref[0, :, base:base + TILE] = acc
        st = st + jnp.sum(acc, axis=1, keepdims=True)
        qt = qt + jnp.sum(acc * acc, axis=1, keepdims=True)
    s_ref[0] = st
    q_ref[0] = qt


def _conv_stats(x, scale, shift, wk, *, H, W, apply_act):
    N, C, HW = x.shape
    Cout = wk.shape[1]
    TILE = 512
    PADL = 128
    SW = 2 * PADL + HW - 2 * W
    body = functools.partial(_conv_stats_kernel, H=H, W=W, TILE=TILE,
                             PADL=PADL, apply_act=apply_act)
    return pl.pallas_call(
        body,
        out_shape=(jax.ShapeDtypeStruct((N, Cout, HW), jnp.float32),
                   jax.ShapeDtypeStruct((N, Cout, 1), jnp.float32),
                   jax.ShapeDtypeStruct((N, Cout, 1), jnp.float32)),
        grid=(N,),
        in_specs=[
            pl.BlockSpec((1, C, HW), lambda n: (n, 0, 0)),
            pl.BlockSpec((C, 1), lambda n: (0, 0)),
            pl.BlockSpec((C, 1), lambda n: (0, 0)),
            pl.BlockSpec((3, Cout, 3 * C), lambda n: (0, 0, 0)),
        ],
        out_specs=(
            pl.BlockSpec((1, Cout, HW), lambda n: (n, 0, 0)),
            pl.BlockSpec((1, Cout, 1), lambda n: (n, 0, 0)),
            pl.BlockSpec((1, Cout, 1), lambda n: (n, 0, 0)),
        ),
        scratch_shapes=[
            pltpu.VMEM((C, PADL + HW + PADL), jnp.bfloat16),
            pltpu.VMEM((3 * C, SW), jnp.bfloat16),
        ],
        compiler_params=pltpu.CompilerParams(
            dimension_semantics=("parallel",)),
    )(x, scale, shift, wk)


def _bn_relu_kernel(x_ref, scale_ref, shift_ref, o_ref):
    o_ref[0] = jnp.maximum(
        x_ref[0] * scale_ref[...] + shift_ref[...], 0.0)


def _bn_relu(x, scale, shift):
    N, C, HW = x.shape
    return pl.pallas_call(
        _bn_relu_kernel,
        out_shape=jax.ShapeDtypeStruct((N, C, HW), jnp.float32),
        grid=(N,),
        in_specs=[
            pl.BlockSpec((1, C, HW), lambda n: (n, 0, 0)),
            pl.BlockSpec((C, 1), lambda n: (0, 0)),
            pl.BlockSpec((C, 1), lambda n: (0, 0)),
        ],
        out_specs=pl.BlockSpec((1, C, HW), lambda n: (n, 0, 0)),
        compiler_params=pltpu.CompilerParams(
            dimension_semantics=("parallel",)),
    )(x, scale, shift)


def _bn_affine(s, q, count, gamma, beta):
    sm = jnp.sum(s[:, :, 0], axis=0)
    sq = jnp.sum(q[:, :, 0], axis=0)
    mean = sm / count
    var = jnp.maximum(sq / count - mean * mean, 0.0)
    scale = gamma[0] / jnp.sqrt(var + EPS)
    shift = beta[0] - mean * scale
    return scale[:, None], shift[:, None]


def _prep_w(w):
    return jnp.transpose(w, (1, 3, 0, 2)).reshape(
        3, w.shape[3], 3 * w.shape[2]).astype(jnp.bfloat16)


@jax.jit
def _unet_down(x_nchw, w1, g1, bt1, w2, g2, bt2):
    N, C, H, W = x_nchw.shape
    HW = H * W
    count = N * HW
    x = x_nchw.reshape(N, C, HW)

    wk1 = _prep_w(w1)
    wk2 = _prep_w(w2)
    ones = jnp.ones((C, 1), jnp.float32)
    zeros = jnp.zeros((C, 1), jnp.float32)

    y1, s1, q1 = _conv_stats(x, ones, zeros, wk1, H=H, W=W, apply_act=False)
    scale1, shift1 = _bn_affine(s1, q1, count, g1, bt1)

    y2, s2, q2 = _conv_stats(y1, scale1, shift1, wk2, H=H, W=W,
                             apply_act=True)
    scale2, shift2 = _bn_affine(s2, q2, count, g2, bt2)

    out = _bn_relu(y2, scale2, shift2)
    return out.reshape(N, C, H, W)


def kernel(x, w1, b1, g1, bt1, w2, b2, g2, bt2):
    return _unet_down(x, w1, g1, bt1, w2, g2, bt2)


# speedup vs baseline: 1.2846x; 1.2846x over previous
import functools

import jax
import jax.numpy as jnp
from jax import lax
from jax.experimental import pallas as pl
from jax.experimental.pallas import tpu as pltpu

EPS = 1e-5


def _conv_stats_kernel(x_ref, scale_ref, shift_ref, w_ref,
                       y_ref, s_ref, q_ref,
                       xpad, col9, *, H, W, TILE, CHUNK, PADL, apply_act):
    C = x_ref.shape[1]
    HW = H * W

    x = x_ref[0]
    if apply_act:
        x = jnp.maximum(x.astype(jnp.float32) * scale_ref[...]
                        + shift_ref[...], 0.0)
    xb = x.astype(jnp.bfloat16)

    xpad[:, :PADL] = jnp.zeros((C, PADL), jnp.bfloat16)
    xpad[:, PADL + HW:] = jnp.zeros((C, PADL), jnp.bfloat16)
    xpad[:, PADL:PADL + HW] = xb

    j = lax.broadcasted_iota(jnp.int32, (1, CHUNK), 1) % W
    m0 = (j != 0).astype(jnp.bfloat16)
    m2 = (j != W - 1).astype(jnp.bfloat16)

    for c0 in range(0, HW, CHUNK):
        for kh in range(3):
            for kw in range(3):
                tap = kh * 3 + kw
                s = PADL + (kh - 1) * W + (kw - 1)
                src = xpad[:, c0 + s:c0 + s + CHUNK]
                if kw == 0:
                    src = src * m0
                elif kw == 2:
                    src = src * m2
                col9[tap * C:(tap + 1) * C, c0:c0 + CHUNK] = src

    st = jnp.zeros((C, 1), jnp.float32)
    qt = jnp.zeros((C, 1), jnp.float32)
    for t in range(HW // TILE):
        base = t * TILE
        acc = jnp.dot(w_ref[...], col9[:, base:base + TILE],
                      preferred_element_type=jnp.float32)
        y_ref[0, :, base:base + TILE] = acc.astype(y_ref.dtype)
        st = st + jnp.sum(acc, axis=1, keepdims=True)
        qt = qt + jnp.sum(acc * acc, axis=1, keepdims=True)
    s_ref[0] = st
    q_ref[0] = qt


def _conv_stats(x, scale, shift, wk, *, H, W, apply_act, out_dtype):
    N, C, HW = x.shape
    Cout = wk.shape[0]
    TILE = 512
    CHUNK = 1024
    PADL = 128
    body = functools.partial(_conv_stats_kernel, H=H, W=W, TILE=TILE,
                             CHUNK=CHUNK, PADL=PADL, apply_act=apply_act)
    return pl.pallas_call(
        body,
        out_shape=(jax.ShapeDtypeStruct((N, Cout, HW), out_dtype),
                   jax.ShapeDtypeStruct((N, Cout, 1), jnp.float32),
                   jax.ShapeDtypeStruct((N, Cout, 1), jnp.float32)),
        grid=(N,),
        in_specs=[
            pl.BlockSpec((1, C, HW), lambda n: (n, 0, 0)),
            pl.BlockSpec((C, 1), lambda n: (0, 0)),
            pl.BlockSpec((C, 1), lambda n: (0, 0)),
            pl.BlockSpec((Cout, 9 * C), lambda n: (0, 0)),
        ],
        out_specs=(
            pl.BlockSpec((1, Cout, HW), lambda n: (n, 0, 0)),
            pl.BlockSpec((1, Cout, 1), lambda n: (n, 0, 0)),
            pl.BlockSpec((1, Cout, 1), lambda n: (n, 0, 0)),
        ),
        scratch_shapes=[
            pltpu.VMEM((C, PADL + HW + PADL), jnp.bfloat16),
            pltpu.VMEM((9 * C, HW), jnp.bfloat16),
        ],
        compiler_params=pltpu.CompilerParams(
            dimension_semantics=("parallel",)),
    )(x, scale, shift, wk)


def _bn_relu_kernel(x_ref, scale_ref, shift_ref, o_ref):
    o_ref[0] = jnp.maximum(
        x_ref[0].astype(jnp.float32) * scale_ref[...] + shift_ref[...], 0.0)


def _bn_relu(x, scale, shift):
    N, C, HW = x.shape
    return pl.pallas_call(
        _bn_relu_kernel,
        out_shape=jax.ShapeDtypeStruct((N, C, HW), jnp.float32),
        grid=(N,),
        in_specs=[
            pl.BlockSpec((1, C, HW), lambda n: (n, 0, 0)),
            pl.BlockSpec((C, 1), lambda n: (0, 0)),
            pl.BlockSpec((C, 1), lambda n: (0, 0)),
        ],
        out_specs=pl.BlockSpec((1, C, HW), lambda n: (n, 0, 0)),
        compiler_params=pltpu.CompilerParams(
            dimension_semantics=("parallel",)),
    )(x, scale, shift)


def _bn_affine(s, q, count, gamma, beta):
    sm = jnp.sum(s[:, :, 0], axis=0)
    sq = jnp.sum(q[:, :, 0], axis=0)
    mean = sm / count
    var = jnp.maximum(sq / count - mean * mean, 0.0)
    scale = gamma[0] / jnp.sqrt(var + EPS)
    shift = beta[0] - mean * scale
    return scale[:, None], shift[:, None]


def _prep_w(w):
    return jnp.transpose(w, (3, 0, 1, 2)).reshape(
        w.shape[3], 9 * w.shape[2]).astype(jnp.bfloat16)


@jax.jit
def _unet_down(x_nchw, w1, g1, bt1, w2, g2, bt2):
    N, C, H, W = x_nchw.shape
    HW = H * W
    count = N * HW
    x = x_nchw.reshape(N, C, HW)

    wk1 = _prep_w(w1)
    wk2 = _prep_w(w2)
    ones = jnp.ones((C, 1), jnp.float32)
    zeros = jnp.zeros((C, 1), jnp.float32)

    y1, s1, q1 = _conv_stats(x, ones, zeros, wk1, H=H, W=W,
                             apply_act=False, out_dtype=jnp.bfloat16)
    scale1, shift1 = _bn_affine(s1, q1, count, g1, bt1)

    y2, s2, q2 = _conv_stats(y1, scale1, shift1, wk2, H=H, W=W,
                             apply_act=True, out_dtype=jnp.bfloat16)
    scale2, shift2 = _bn_affine(s2, q2, count, g2, bt2)

    out = _bn_relu(y2, scale2, shift2)
    return out.reshape(N, C, H, W)


def kernel(x, w1, b1, g1, bt1, w2, b2, g2, bt2):
    return _unet_down(x, w1, g1, bt1, w2, g2, bt2)


# speedup vs baseline: 1.8325x; 1.4265x over previous
import functools

import jax
import jax.numpy as jnp
from jax import lax
from jax.experimental import pallas as pl
from jax.experimental.pallas import tpu as pltpu

EPS = 1e-5


def _affine(s_ref, q_ref, g_ref, bt_ref, inv_count):
    mean = s_ref[...] * inv_count
    var = jnp.maximum(q_ref[...] * inv_count - mean * mean, 0.0)
    scale = g_ref[...] / jnp.sqrt(var + EPS)
    shift = bt_ref[...] - mean * scale
    return scale, shift


def _conv_phase(xb, w_ref, y_scr, n, s_ref, q_ref, xpad, col9,
                *, H, W, TILE, CHUNK, PADL):
    C, HW = xb.shape

    xpad[:, :PADL] = jnp.zeros((C, PADL), jnp.bfloat16)
    xpad[:, PADL + HW:] = jnp.zeros((C, PADL), jnp.bfloat16)
    xpad[:, PADL:PADL + HW] = xb

    j = lax.broadcasted_iota(jnp.int32, (1, CHUNK), 1) % W
    m0 = (j != 0).astype(jnp.bfloat16)
    m2 = (j != W - 1).astype(jnp.bfloat16)

    for c0 in range(0, HW, CHUNK):
        for kh in range(3):
            for kw in range(3):
                tap = kh * 3 + kw
                s = PADL + (kh - 1) * W + (kw - 1)
                src = xpad[:, c0 + s:c0 + s + CHUNK]
                if kw == 0:
                    src = src * m0
                elif kw == 2:
                    src = src * m2
                col9[tap * C:(tap + 1) * C, c0:c0 + CHUNK] = src

    st = jnp.zeros((C, 1), jnp.float32)
    qt = jnp.zeros((C, 1), jnp.float32)
    for t in range(HW // TILE):
        base = t * TILE
        acc = jnp.dot(w_ref[...], col9[:, base:base + TILE],
                      preferred_element_type=jnp.float32)
        y_scr[n, :, base:base + TILE] = acc.astype(jnp.bfloat16)
        st = st + jnp.sum(acc, axis=1, keepdims=True)
        qt = qt + jnp.sum(acc * acc, axis=1, keepdims=True)
    s_ref[...] += st
    q_ref[...] += qt


def _fused_kernel(x_ref, w1_ref, w2_ref, g1_ref, bt1_ref, g2_ref, bt2_ref,
                  o_ref,
                  y1, y2, col9, xpad, s1, q1, s2, q2,
                  *, H, W, TILE, CHUNK, PADL, N):
    p = pl.program_id(0)
    n = pl.program_id(1)
    inv_count = 1.0 / (N * H * W)
    conv = functools.partial(_conv_phase, H=H, W=W, TILE=TILE, CHUNK=CHUNK,
                             PADL=PADL)

    @pl.when(jnp.logical_and(p == 0, n == 0))
    def _init():
        s1[...] = jnp.zeros_like(s1)
        q1[...] = jnp.zeros_like(q1)
        s2[...] = jnp.zeros_like(s2)
        q2[...] = jnp.zeros_like(q2)

    @pl.when(p == 0)
    def _phase0():
        xb = jnp.transpose(x_ref[0]).astype(jnp.bfloat16)
        conv(xb, w1_ref, y1, n, s1, q1, xpad, col9)

    @pl.when(p == 1)
    def _phase1():
        scale, shift = _affine(s1, q1, g1_ref, bt1_ref, inv_count)
        a = jnp.maximum(
            y1[n].astype(jnp.float32) * scale + shift, 0.0)
        conv(a.astype(jnp.bfloat16), w2_ref, y2, n, s2, q2, xpad, col9)

    @pl.when(p == 2)
    def _phase2():
        scale, shift = _affine(s2, q2, g2_ref, bt2_ref, inv_count)
        v = jnp.maximum(
            y2[n].astype(jnp.float32) * scale + shift, 0.0)
        o_ref[0] = jnp.transpose(v)


def _prep_w(w):
    return jnp.transpose(w, (3, 0, 1, 2)).reshape(
        w.shape[3], 9 * w.shape[2]).astype(jnp.bfloat16)


@jax.jit
def _unet_down(x_nchw, w1, g1, bt1, w2, g2, bt2):
    N, C, H, W = x_nchw.shape
    HW = H * W
    TILE = 512
    CHUNK = 1024
    PADL = 128

    xv = jnp.transpose(x_nchw, (0, 2, 3, 1)).reshape(N, HW, C)
    wk1 = _prep_w(w1)
    wk2 = _prep_w(w2)
    g1t, bt1t = g1.reshape(C, 1), bt1.reshape(C, 1)
    g2t, bt2t = g2.reshape(C, 1), bt2.reshape(C, 1)

    body = functools.partial(_fused_kernel, H=H, W=W, TILE=TILE,
                             CHUNK=CHUNK, PADL=PADL, N=N)
    last = N - 1
    out = pl.pallas_call(
        body,
        out_shape=jax.ShapeDtypeStruct((N, HW, C), jnp.float32),
        grid=(3, N),
        in_specs=[
            pl.BlockSpec((1, HW, C),
                         lambda p, n: (jnp.where(p == 0, n, last), 0, 0)),
            pl.BlockSpec((C, 9 * C), lambda p, n: (0, 0)),
            pl.BlockSpec((C, 9 * C), lambda p, n: (0, 0)),
            pl.BlockSpec((C, 1), lambda p, n: (0, 0)),
            pl.BlockSpec((C, 1), lambda p, n: (0, 0)),
            pl.BlockSpec((C, 1), lambda p, n: (0, 0)),
            pl.BlockSpec((C, 1), lambda p, n: (0, 0)),
        ],
        out_specs=pl.BlockSpec((1, HW, C),
                               lambda p, n: (jnp.where(p == 2, n, 0), 0, 0)),
        scratch_shapes=[
            pltpu.VMEM((N, C, HW), jnp.bfloat16),
            pltpu.VMEM((N, C, HW), jnp.bfloat16),
            pltpu.VMEM((9 * C, HW), jnp.bfloat16),
            pltpu.VMEM((C, PADL + HW + PADL), jnp.bfloat16),
            pltpu.VMEM((C, 1), jnp.float32),
            pltpu.VMEM((C, 1), jnp.float32),
            pltpu.VMEM((C, 1), jnp.float32),
            pltpu.VMEM((C, 1), jnp.float32),
        ],
        compiler_params=pltpu.CompilerParams(
            dimension_semantics=("arbitrary", "arbitrary")),
    )(xv, wk1, wk2, g1t, bt1t, g2t, bt2t)

    return jnp.transpose(out.reshape(N, H, W, C), (0, 3, 1, 2))


def kernel(x, w1, b1, g1, bt1, w2, b2, g2, bt2):
    return _unet_down(x, w1, g1, bt1, w2, g2, bt2)


# speedup vs baseline: 1.8858x; 1.0291x over previous
import functools

import jax
import jax.numpy as jnp
from jax import lax
from jax.experimental import pallas as pl
from jax.experimental.pallas import tpu as pltpu

EPS = 1e-5


def _affine(s_ref, q_ref, g_ref, bt_ref, inv_count):
    mean = s_ref[...] * inv_count
    var = jnp.maximum(q_ref[...] * inv_count - mean * mean, 0.0)
    scale = g_ref[...] / jnp.sqrt(var + EPS)
    shift = bt_ref[...] - mean * scale
    return scale, shift


def _conv_phase(xb, w_ref, y_scr, n, s_ref, q_ref, xpad, xpad0, xpad2, col9,
                *, H, W, TILE, CHUNK, PADL):
    C, HW = xb.shape
    PW = PADL + HW + PADL

    xpad[:, :PADL] = jnp.zeros((C, PADL), jnp.bfloat16)
    xpad[:, PADL + HW:] = jnp.zeros((C, PADL), jnp.bfloat16)
    xpad[:, PADL:PADL + HW] = xb

    cw = lax.broadcasted_iota(jnp.int32, (C, PW), 1) % W
    z = jnp.zeros((C, PW), jnp.bfloat16)
    xpad0[...] = jnp.where(cw == (PADL - 1) % W, z, xpad[...])
    xpad2[...] = jnp.where(cw == PADL % W, z, xpad[...])

    srcs = (xpad0, xpad, xpad2)
    for c0 in range(0, HW, CHUNK):
        for kh in range(3):
            for kw in range(3):
                tap = kh * 3 + kw
                s = PADL + (kh - 1) * W + (kw - 1)
                col9[tap * C:(tap + 1) * C, c0:c0 + CHUNK] = \
                    srcs[kw][:, c0 + s:c0 + s + CHUNK]

    st = jnp.zeros((C, 1), jnp.float32)
    qt = jnp.zeros((C, 1), jnp.float32)
    for t in range(HW // TILE):
        base = t * TILE
        acc = jnp.dot(w_ref[...], col9[:, base:base + TILE],
                      preferred_element_type=jnp.float32)
        y_scr[n, :, base:base + TILE] = acc.astype(jnp.bfloat16)
        st = st + jnp.sum(acc, axis=1, keepdims=True)
        qt = qt + jnp.sum(acc * acc, axis=1, keepdims=True)
    s_ref[...] += st
    q_ref[...] += qt


def _fused_kernel(x_ref, w1_ref, w2_ref, g1_ref, bt1_ref, g2_ref, bt2_ref,
                  o_ref,
                  y1, y2, col9, xpad, xpad0, xpad2, s1, q1, s2, q2,
                  *, H, W, TILE, CHUNK, PADL, N):
    p = pl.program_id(0)
    n = pl.program_id(1)
    inv_count = 1.0 / (N * H * W)
    conv = functools.partial(_conv_phase, H=H, W=W, TILE=TILE, CHUNK=CHUNK,
                             PADL=PADL)

    @pl.when(jnp.logical_and(p == 0, n == 0))
    def _init():
        s1[...] = jnp.zeros_like(s1)
        q1[...] = jnp.zeros_like(q1)
        s2[...] = jnp.zeros_like(s2)
        q2[...] = jnp.zeros_like(q2)

    @pl.when(p == 0)
    def _phase0():
        xb = jnp.transpose(x_ref[0]).astype(jnp.bfloat16)
        conv(xb, w1_ref, y1, n, s1, q1, xpad, xpad0, xpad2, col9)

    @pl.when(p == 1)
    def _phase1():
        scale, shift = _affine(s1, q1, g1_ref, bt1_ref, inv_count)
        a = jnp.maximum(
            y1[n].astype(jnp.float32) * scale + shift, 0.0)
        conv(a.astype(jnp.bfloat16), w2_ref, y2, n, s2, q2,
             xpad, xpad0, xpad2, col9)

    @pl.when(p == 2)
    def _phase2():
        scale, shift = _affine(s2, q2, g2_ref, bt2_ref, inv_count)
        v = jnp.maximum(
            y2[n].astype(jnp.float32) * scale + shift, 0.0)
        o_ref[0] = jnp.transpose(v)


def _prep_w(w):
    return jnp.transpose(w, (3, 0, 1, 2)).reshape(
        w.shape[3], 9 * w.shape[2]).astype(jnp.bfloat16)


@jax.jit
def _unet_down(x_nchw, w1, g1, bt1, w2, g2, bt2):
    N, C, H, W = x_nchw.shape
    HW = H * W
    TILE = 512
    CHUNK = 1024
    PADL = 128

    xv = jnp.transpose(x_nchw, (0, 2, 3, 1)).reshape(N, HW, C)
    wk1 = _prep_w(w1)
    wk2 = _prep_w(w2)
    g1t, bt1t = g1.reshape(C, 1), bt1.reshape(C, 1)
    g2t, bt2t = g2.reshape(C, 1), bt2.reshape(C, 1)

    body = functools.partial(_fused_kernel, H=H, W=W, TILE=TILE,
                             CHUNK=CHUNK, PADL=PADL, N=N)
    last = N - 1
    out = pl.pallas_call(
        body,
        out_shape=jax.ShapeDtypeStruct((N, HW, C), jnp.float32),
        grid=(3, N),
        in_specs=[
            pl.BlockSpec((1, HW, C),
                         lambda p, n: (jnp.where(p == 0, n, last), 0, 0)),
            pl.BlockSpec((C, 9 * C), lambda p, n: (0, 0)),
            pl.BlockSpec((C, 9 * C), lambda p, n: (0, 0)),
            pl.BlockSpec((C, 1), lambda p, n: (0, 0)),
            pl.BlockSpec((C, 1), lambda p, n: (0, 0)),
            pl.BlockSpec((C, 1), lambda p, n: (0, 0)),
            pl.BlockSpec((C, 1), lambda p, n: (0, 0)),
        ],
        out_specs=pl.BlockSpec((1, HW, C),
                               lambda p, n: (jnp.where(p == 2, n, 0), 0, 0)),
        scratch_shapes=[
            pltpu.VMEM((N, C, HW), jnp.bfloat16),
            pltpu.VMEM((N, C, HW), jnp.bfloat16),
            pltpu.VMEM((9 * C, HW), jnp.bfloat16),
            pltpu.VMEM((C, PADL + HW + PADL), jnp.bfloat16),
            pltpu.VMEM((C, PADL + HW + PADL), jnp.bfloat16),
            pltpu.VMEM((C, PADL + HW + PADL), jnp.bfloat16),
            pltpu.VMEM((C, 1), jnp.float32),
            pltpu.VMEM((C, 1), jnp.float32),
            pltpu.VMEM((C, 1), jnp.float32),
            pltpu.VMEM((C, 1), jnp.float32),
        ],
        compiler_params=pltpu.CompilerParams(
            dimension_semantics=("arbitrary", "arbitrary")),
    )(xv, wk1, wk2, g1t, bt1t, g2t, bt2t)

    return jnp.transpose(out.reshape(N, H, W, C), (0, 3, 1, 2))


def kernel(x, w1, b1, g1, bt1, w2, b2, g2, bt2):
    return _unet_down(x, w1, g1, bt1, w2, g2, bt2)
